# Initial kernel scaffold; baseline (speedup 1.0000x reference)
import jax
import jax.numpy as jnp
from jax.experimental import pallas as pl


def kernel(x, edge_index, edge_attr, We1a, be1a, We2a, be2a, Wn1a, bn1a, Wn2a, bn2a, We1b, be1b, We2b, be2b, Wn1b, bn1b, Wn2b, bn2b, Wp1, bp1, Wp2, bp2):
    raise NotImplementedError("write your pallas kernel here")


# speedup vs baseline: 1.9091x; 1.9091x over previous
import functools

import jax
import jax.numpy as jnp
from jax import lax
from jax.experimental import pallas as pl
from jax.experimental.pallas import tpu as pltpu
from jax.experimental.pallas import tpu_sc as plsc

NC = 2
NS = 16
NW = NC * NS

CH = 80


def _sc_gather_pair(table, src, dst):
    n, d = table.shape
    e = src.shape[0]
    per_w = e // NW
    nch = per_w // CH
    mesh = plsc.VectorSubcoreMesh(core_axis_name="c", subcore_axis_name="s")

    @functools.partial(
        pl.kernel, mesh=mesh,
        out_type=[jax.ShapeDtypeStruct((e, d), jnp.float32),
                  jax.ShapeDtypeStruct((e, d), jnp.float32)],
        scratch_types=[pltpu.VMEM((CH,), jnp.int32),
                       pltpu.VMEM((CH,), jnp.int32),
                       pltpu.VMEM((CH, d), jnp.float32),
                       pltpu.VMEM((CH, d), jnp.float32),
                       pltpu.SemaphoreType.DMA,
                       pltpu.SemaphoreType.DMA],
    )
    def k(table_hbm, src_hbm, dst_hbm, outs_hbm, outd_hbm,
          idx_s, idx_d, rows_s, rows_d, sem_s, sem_d):
        wid = lax.axis_index("s") * NC + lax.axis_index("c")
        base = wid * per_w

        def body(i, carry):
            off = base + i * CH
            pltpu.sync_copy(src_hbm.at[pl.ds(off, CH)], idx_s)
            pltpu.sync_copy(dst_hbm.at[pl.ds(off, CH)], idx_d)
            cs = pltpu.async_copy(table_hbm.at[idx_s], rows_s, sem_s)
            cd = pltpu.async_copy(table_hbm.at[idx_d], rows_d, sem_d)
            cs.wait()
            cd.wait()
            pltpu.sync_copy(rows_s, outs_hbm.at[pl.ds(off, CH)])
            pltpu.sync_copy(rows_d, outd_hbm.at[pl.ds(off, CH)])
            return carry

        lax.fori_loop(0, nch, body, 0)

    return k(table, src, dst)


def _sc_segment_sum(vals, dst, n, zeros_init, zeros_cnt, ones_blk):
    e, feat = vals.shape
    half = feat // NC
    per_t = e // NS
    nch = per_t // CH
    hn = n // 2
    hz = hn + 128
    zt = hz // NS
    pt = hn // NS
    assert pt % 8 == 0 and zt % 8 == 0 and hn % NS == 0 and hz % NS == 0
    mesh = plsc.VectorSubcoreMesh(core_axis_name="c", subcore_axis_name="s")

    assert pt % CH == 0
    @functools.partial(
        pl.kernel, mesh=mesh,
        out_type=[jax.ShapeDtypeStruct((n, feat), jnp.float32),
                  jax.ShapeDtypeStruct((n, 16), jnp.float32)],
        scratch_types=[pltpu.VMEM((CH,), jnp.int32),
                       pltpu.VMEM((CH,), jnp.int32),
                       pltpu.VMEM((CH, half), jnp.float32),
                       pltpu.VMEM((CH, 16), jnp.float32),
                       pltpu.VMEM((8, half), jnp.float32),
                       pltpu.VMEM((8, 16), jnp.float32),
                       pltpu.VMEM((CH, 16), jnp.float32),
                       pltpu.VMEM_SHARED((hz, half), jnp.float32),
                       pltpu.VMEM_SHARED((hz, 16), jnp.float32)],
    )
    def k(vals_hbm, dst_hbm, z_hbm, zc_hbm, ones_hbm, agg_hbm, cnt_hbm,
          idx_v, idx_m, rows_v, ones_v, zbuf, cbuf, cnt_b, agg_sh, cnt_sh):
        c = lax.axis_index("c")
        s = lax.axis_index("s")
        coff = pl.multiple_of(c * half, half)

        pltpu.sync_copy(z_hbm, zbuf)
        pltpu.sync_copy(zc_hbm, cbuf)
        pltpu.sync_copy(ones_hbm, ones_v)

        for p in range(2):
            base = p * hn
            def zloop(i, carry):
                pltpu.sync_copy(zbuf, agg_sh.at[pl.ds(s * zt + i * 8, 8)])
                pltpu.sync_copy(cbuf, cnt_sh.at[pl.ds(s * zt + i * 8, 8)])
                return carry

            lax.fori_loop(0, zt // 8, zloop, 0)
            plsc.subcore_barrier()

            def body(i, carry):
                off = s * per_t + i * CH
                pltpu.sync_copy(dst_hbm.at[pl.ds(off, CH)], idx_v)
                pltpu.sync_copy(vals_hbm.at[pl.ds(off, CH),
                                            pl.ds(coff, half)], rows_v)
                for j in range(CH // 16):
                    v = idx_v[pl.ds(j * 16, 16)] - base
                    ok = jnp.logical_and(v >= 0, v < hn)
                    idx_m[pl.ds(j * 16, 16)] = jnp.where(ok, v, hn)
                pltpu.sync_copy(rows_v, agg_sh.at[idx_m], add=True)

                @pl.when(c == 0)
                def _cnt():
                    pltpu.sync_copy(ones_v, cnt_sh.at[idx_m], add=True)

                return carry

            lax.fori_loop(0, nch, body, 0)
            plsc.subcore_barrier()

            def oloop(i, carry):
                r = s * pt + i * CH
                pltpu.sync_copy(agg_sh.at[pl.ds(r, CH)], rows_v)
                pltpu.sync_copy(rows_v,
                                agg_hbm.at[pl.ds(base + r, CH),
                                           pl.ds(coff, half)])

                @pl.when(c == 0)
                def _oc():
                    pltpu.sync_copy(cnt_sh.at[pl.ds(r, CH)], cnt_b)
                    pltpu.sync_copy(cnt_b, cnt_hbm.at[pl.ds(base + r, CH)])

                return carry

            lax.fori_loop(0, pt // CH, oloop, 0)
            plsc.subcore_barrier()

    return k(vals, dst, zeros_init, zeros_cnt, ones_blk)


def _e1_body(xs, xd, ea, w1s, w1d, w1e, b1, w2, b2, out):
    h = xs[...] @ w1s[...] + xd[...] @ w1d[...] + ea[...] @ w1e[...] + b1[...]
    h = jnp.maximum(h, 0.0)
    out[...] = h @ w2[...] + b2[...] + h


def _tc_edge1(xs, xd, ea, w1s, w1d, w1e, b1, w2, b2, block):
    e, d = xs.shape
    de = ea.shape[1]
    l = w2.shape[0]
    full = lambda *shape: pl.BlockSpec(shape, lambda i: (0,) * len(shape))
    return pl.pallas_call(
        _e1_body,
        grid=(e // block,),
        in_specs=[pl.BlockSpec((block, d), lambda i: (i, 0)),
                  pl.BlockSpec((block, d), lambda i: (i, 0)),
                  pl.BlockSpec((block, de), lambda i: (i, 0)),
                  full(d, l), full(d, l), full(de, l), full(1, l),
                  full(l, l), full(1, l)],
        out_specs=pl.BlockSpec((block, l), lambda i: (i, 0)),
        out_shape=jax.ShapeDtypeStruct((e, l), jnp.float32),
    )(xs, xd, ea, w1s, w1d, w1e, b1, w2, b2)


def _node_body(x, aggs, cnt, w1x, w1a, b1, w2, b2, out):
    agg = aggs[...] / jnp.maximum(cnt[...][:, 0:1], 1.0)
    h = x[...] @ w1x[...] + agg @ w1a[...] + b1[...]
    h = jnp.maximum(h, 0.0)
    out[...] = h @ w2[...] + b2[...] + h


def _tc_node(x, agg_sum, cnt, w1x, w1a, b1, w2, b2, block):
    n, d = x.shape
    l = w2.shape[0]
    full = lambda *shape: pl.BlockSpec(shape, lambda i: (0,) * len(shape))
    return pl.pallas_call(
        _node_body,
        grid=(n // block,),
        in_specs=[pl.BlockSpec((block, d), lambda i: (i, 0)),
                  pl.BlockSpec((block, l), lambda i: (i, 0)),
                  pl.BlockSpec((block, 16), lambda i: (i, 0)),
                  full(d, l), full(l, l), full(1, l),
                  full(l, l), full(1, l)],
        out_specs=pl.BlockSpec((block, l), lambda i: (i, 0)),
        out_shape=jax.ShapeDtypeStruct((n, l), jnp.float32),
    )(x, agg_sum, cnt, w1x, w1a, b1, w2, b2)


def _e2_body(s1, d1, e1, w1s, w1d, w1e, b1, w2, b2, wp1, bp1, wp2r, bp2, out):
    h = s1[...] @ w1s[...] + d1[...] @ w1d[...] + e1[...] @ w1e[...] + b1[...]
    h = jnp.maximum(h, 0.0)
    e2 = h @ w2[...] + b2[...] + h
    t = e2 @ wp1[...] + bp1[...]
    p = jnp.where(t >= 0.0, t, t * 0.01)
    out[...] = jnp.sum(p * wp2r[...], axis=1, keepdims=True) + bp2[...]


def _tc_edge2(s1, d1, e1, w1s, w1d, w1e, b1, w2, b2, wp1, bp1, wp2r, bp2,
              block):
    e, l = e1.shape
    full = lambda *shape: pl.BlockSpec(shape, lambda i: (0,) * len(shape))
    return pl.pallas_call(
        _e2_body,
        grid=(e // block,),
        in_specs=[pl.BlockSpec((block, l), lambda i: (i, 0)),
                  pl.BlockSpec((block, l), lambda i: (i, 0)),
                  pl.BlockSpec((block, l), lambda i: (i, 0)),
                  full(l, l), full(l, l), full(l, l), full(1, l),
                  full(l, l), full(1, l),
                  full(l, l), full(1, l), full(1, l), full(1, 1)],
        out_specs=pl.BlockSpec((block, 1), lambda i: (i, 0)),
        out_shape=jax.ShapeDtypeStruct((e, 1), jnp.float32),
    )(s1, d1, e1, w1s, w1d, w1e, b1, w2, b2, wp1, bp1, wp2r, bp2)


def kernel(x, edge_index, edge_attr, We1a, be1a, We2a, be2a, Wn1a, bn1a,
           Wn2a, bn2a, We1b, be1b, We2b, be2b, Wn1b, bn1b, Wn2b, bn2b,
           Wp1, bp1, Wp2, bp2):
    n, d = x.shape
    l = We2a.shape[0]
    src = edge_index[0].astype(jnp.int32)
    dst = edge_index[1].astype(jnp.int32)

    xs, xd = _sc_gather_pair(x, src, dst)
    e1 = _tc_edge1(xs, xd, edge_attr,
                   We1a[:d], We1a[d:2 * d], We1a[2 * d:],
                   be1a.reshape(1, l), We2a, be2a.reshape(1, l),
                   block=2000)

    npad = ((n + 255) // 256) * 256
    zeros_init = jnp.zeros((8, l // NC), jnp.float32)
    zeros_cnt = jnp.zeros((8, 16), jnp.float32)
    ones_blk = jnp.ones((CH, 16), jnp.float32)
    agg_pad, cnt_pad = _sc_segment_sum(e1, dst, npad, zeros_init, zeros_cnt,
                                       ones_blk)
    agg_sum = agg_pad[:n]
    cnt = cnt_pad[:n]

    x1 = _tc_node(x, agg_sum, cnt,
                  Wn1a[:d], Wn1a[d:], bn1a.reshape(1, l),
                  Wn2a, bn2a.reshape(1, l), block=1000)

    xs1, xd1 = _sc_gather_pair(x1, src, dst)
    out = _tc_edge2(xs1, xd1, e1,
                    We1b[:l], We1b[l:2 * l], We1b[2 * l:],
                    be1b.reshape(1, l), We2b, be2b.reshape(1, l),
                    Wp1, bp1.reshape(1, l),
                    Wp2.reshape(1, l), bp2.reshape(1, 1),
                    block=2000)
    return out[:, 0]


# speedup vs baseline: 1.9530x; 1.0230x over previous
import functools

import jax
import jax.numpy as jnp
from jax import lax
from jax.experimental import pallas as pl
from jax.experimental.pallas import tpu as pltpu
from jax.experimental.pallas import tpu_sc as plsc

NC = 2
NS = 16
NW = NC * NS

CH = 80


def _sc_gather_pair2(table_s, table_d, src, dst):
    n, d = table_s.shape
    e = src.shape[0]
    per_w = e // NW
    nch = per_w // CH
    mesh = plsc.VectorSubcoreMesh(core_axis_name="c", subcore_axis_name="s")

    @functools.partial(
        pl.kernel, mesh=mesh,
        out_type=[jax.ShapeDtypeStruct((e, d), jnp.float32),
                  jax.ShapeDtypeStruct((e, d), jnp.float32)],
        scratch_types=[pltpu.VMEM((CH,), jnp.int32),
                       pltpu.VMEM((CH,), jnp.int32),
                       pltpu.VMEM((CH, d), jnp.float32),
                       pltpu.VMEM((CH, d), jnp.float32),
                       pltpu.SemaphoreType.DMA,
                       pltpu.SemaphoreType.DMA],
    )
    def k(tabs_hbm, tabd_hbm, src_hbm, dst_hbm, outs_hbm, outd_hbm,
          idx_s, idx_d, rows_s, rows_d, sem_s, sem_d):
        wid = lax.axis_index("s") * NC + lax.axis_index("c")
        base = wid * per_w

        def body(i, carry):
            off = base + i * CH
            pltpu.sync_copy(src_hbm.at[pl.ds(off, CH)], idx_s)
            pltpu.sync_copy(dst_hbm.at[pl.ds(off, CH)], idx_d)
            cs = pltpu.async_copy(tabs_hbm.at[idx_s], rows_s, sem_s)
            cd = pltpu.async_copy(tabd_hbm.at[idx_d], rows_d, sem_d)
            cs.wait()
            cd.wait()
            pltpu.sync_copy(rows_s, outs_hbm.at[pl.ds(off, CH)])
            pltpu.sync_copy(rows_d, outd_hbm.at[pl.ds(off, CH)])
            return carry

        lax.fori_loop(0, nch, body, 0)

    return k(table_s, table_d, src, dst)


def _sc_segment_sum(vals, dst, n, zeros_init, zeros_cnt, ones_blk):
    e, feat = vals.shape
    half = feat // NC
    per_t = e // NS
    nch = per_t // CH
    hn = n // 2
    hz = hn + 128
    zt = hz // NS
    pt = hn // NS
    assert pt % CH == 0 and zt % 8 == 0 and hn % NS == 0 and hz % NS == 0
    mesh = plsc.VectorSubcoreMesh(core_axis_name="c", subcore_axis_name="s")

    @functools.partial(
        pl.kernel, mesh=mesh,
        out_type=[jax.ShapeDtypeStruct((n, feat), jnp.float32),
                  jax.ShapeDtypeStruct((n, 16), jnp.float32)],
        scratch_types=[pltpu.VMEM((CH,), jnp.int32),
                       pltpu.VMEM((CH,), jnp.int32),
                       pltpu.VMEM((CH, half), jnp.float32),
                       pltpu.VMEM((CH, 16), jnp.float32),
                       pltpu.VMEM((8, half), jnp.float32),
                       pltpu.VMEM((8, 16), jnp.float32),
                       pltpu.VMEM((CH, 16), jnp.float32),
                       pltpu.VMEM_SHARED((hz, half), jnp.float32),
                       pltpu.VMEM_SHARED((hz, 16), jnp.float32)],
    )
    def k(vals_hbm, dst_hbm, z_hbm, zc_hbm, ones_hbm, agg_hbm, cnt_hbm,
          idx_v, idx_m, rows_v, ones_v, zbuf, cbuf, cnt_b, agg_sh, cnt_sh):
        c = lax.axis_index("c")
        s = lax.axis_index("s")
        coff = pl.multiple_of(c * half, half)

        pltpu.sync_copy(z_hbm, zbuf)
        pltpu.sync_copy(zc_hbm, cbuf)
        pltpu.sync_copy(ones_hbm, ones_v)

        for p in range(2):
            base = p * hn
            def zloop(i, carry):
                pltpu.sync_copy(zbuf, agg_sh.at[pl.ds(s * zt + i * 8, 8)])
                pltpu.sync_copy(cbuf, cnt_sh.at[pl.ds(s * zt + i * 8, 8)])
                return carry

            lax.fori_loop(0, zt // 8, zloop, 0)
            plsc.subcore_barrier()

            def body(i, carry):
                off = s * per_t + i * CH
                pltpu.sync_copy(dst_hbm.at[pl.ds(off, CH)], idx_v)
                pltpu.sync_copy(vals_hbm.at[pl.ds(off, CH),
                                            pl.ds(coff, half)], rows_v)
                for j in range(CH // 16):
                    v = idx_v[pl.ds(j * 16, 16)] - base
                    ok = jnp.logical_and(v >= 0, v < hn)
                    idx_m[pl.ds(j * 16, 16)] = jnp.where(ok, v, hn)
                pltpu.sync_copy(rows_v, agg_sh.at[idx_m], add=True)

                @pl.when(c == 0)
                def _cnt():
                    pltpu.sync_copy(ones_v, cnt_sh.at[idx_m], add=True)

                return carry

            lax.fori_loop(0, nch, body, 0)
            plsc.subcore_barrier()

            def oloop(i, carry):
                r = s * pt + i * CH
                pltpu.sync_copy(agg_sh.at[pl.ds(r, CH)], rows_v)
                pltpu.sync_copy(rows_v,
                                agg_hbm.at[pl.ds(base + r, CH),
                                           pl.ds(coff, half)])

                @pl.when(c == 0)
                def _oc():
                    pltpu.sync_copy(cnt_sh.at[pl.ds(r, CH)], cnt_b)
                    pltpu.sync_copy(cnt_b, cnt_hbm.at[pl.ds(base + r, CH)])

                return carry

            lax.fori_loop(0, pt // CH, oloop, 0)
            plsc.subcore_barrier()

    return k(vals, dst, zeros_init, zeros_cnt, ones_blk)


def _e1_body(xs, xd, ea, w1s, w1d, w1e, b1, w2, b2, out):
    h = xs[...] @ w1s[...] + xd[...] @ w1d[...] + ea[...] @ w1e[...] + b1[...]
    h = jnp.maximum(h, 0.0)
    out[...] = h @ w2[...] + b2[...] + h


def _tc_edge1(xs, xd, ea, w1s, w1d, w1e, b1, w2, b2, block):
    e, d = xs.shape
    de = ea.shape[1]
    l = w2.shape[0]
    full = lambda *shape: pl.BlockSpec(shape, lambda i: (0,) * len(shape))
    return pl.pallas_call(
        _e1_body,
        grid=(e // block,),
        in_specs=[pl.BlockSpec((block, d), lambda i: (i, 0)),
                  pl.BlockSpec((block, d), lambda i: (i, 0)),
                  pl.BlockSpec((block, de), lambda i: (i, 0)),
                  full(d, l), full(d, l), full(de, l), full(1, l),
                  full(l, l), full(1, l)],
        out_specs=pl.BlockSpec((block, l), lambda i: (i, 0)),
        out_shape=jax.ShapeDtypeStruct((e, l), jnp.float32),
    )(xs, xd, ea, w1s, w1d, w1e, b1, w2, b2)


def _node_body(x, aggs, cnt, w1x, w1a, b1, w2, b2, wbs, wbd,
               out, outa, outb):
    agg = aggs[...] / jnp.maximum(cnt[...][:, 0:1], 1.0)
    h = x[...] @ w1x[...] + agg @ w1a[...] + b1[...]
    h = jnp.maximum(h, 0.0)
    x1 = h @ w2[...] + b2[...] + h
    out[...] = x1
    outa[...] = x1 @ wbs[...]
    outb[...] = x1 @ wbd[...]


def _tc_node(x, agg_sum, cnt, w1x, w1a, b1, w2, b2, wbs, wbd, block):
    n, d = x.shape
    l = w2.shape[0]
    full = lambda *shape: pl.BlockSpec(shape, lambda i: (0,) * len(shape))
    return pl.pallas_call(
        _node_body,
        grid=(n // block,),
        in_specs=[pl.BlockSpec((block, d), lambda i: (i, 0)),
                  pl.BlockSpec((block, l), lambda i: (i, 0)),
                  pl.BlockSpec((block, 16), lambda i: (i, 0)),
                  full(d, l), full(l, l), full(1, l),
                  full(l, l), full(1, l), full(l, l), full(l, l)],
        out_specs=[pl.BlockSpec((block, l), lambda i: (i, 0))] * 3,
        out_shape=[jax.ShapeDtypeStruct((n, l), jnp.float32)] * 3,
    )(x, agg_sum, cnt, w1x, w1a, b1, w2, b2, wbs, wbd)


def _e2_body(s1, d1, e1, w1e, b1, w2, b2, wp1, bp1, wp2r, bp2, out):
    h = s1[...] + d1[...] + e1[...] @ w1e[...] + b1[...]
    h = jnp.maximum(h, 0.0)
    e2 = h @ w2[...] + b2[...] + h
    t = e2 @ wp1[...] + bp1[...]
    p = jnp.where(t >= 0.0, t, t * 0.01)
    out[...] = jnp.sum(p * wp2r[...], axis=1, keepdims=True) + bp2[...]


def _tc_edge2(s1, d1, e1, w1e, b1, w2, b2, wp1, bp1, wp2r, bp2, block):
    e, l = e1.shape
    full = lambda *shape: pl.BlockSpec(shape, lambda i: (0,) * len(shape))
    return pl.pallas_call(
        _e2_body,
        grid=(e // block,),
        in_specs=[pl.BlockSpec((block, l), lambda i: (i, 0)),
                  pl.BlockSpec((block, l), lambda i: (i, 0)),
                  pl.BlockSpec((block, l), lambda i: (i, 0)),
                  full(l, l), full(1, l),
                  full(l, l), full(1, l),
                  full(l, l), full(1, l), full(1, l), full(1, 1)],
        out_specs=pl.BlockSpec((block, 1), lambda i: (i, 0)),
        out_shape=jax.ShapeDtypeStruct((e, 1), jnp.float32),
    )(s1, d1, e1, w1e, b1, w2, b2, wp1, bp1, wp2r, bp2)


def kernel(x, edge_index, edge_attr, We1a, be1a, We2a, be2a, Wn1a, bn1a,
           Wn2a, bn2a, We1b, be1b, We2b, be2b, Wn1b, bn1b, Wn2b, bn2b,
           Wp1, bp1, Wp2, bp2):
    n, d = x.shape
    l = We2a.shape[0]
    src = edge_index[0].astype(jnp.int32)
    dst = edge_index[1].astype(jnp.int32)

    xs, xd = _sc_gather_pair2(x, x, src, dst)
    e1 = _tc_edge1(xs, xd, edge_attr,
                   We1a[:d], We1a[d:2 * d], We1a[2 * d:],
                   be1a.reshape(1, l), We2a, be2a.reshape(1, l),
                   block=2000)

    npad = ((n + 1279) // 1280) * 1280
    zeros_init = jnp.zeros((8, l // NC), jnp.float32)
    zeros_cnt = jnp.zeros((8, 16), jnp.float32)
    ones_blk = jnp.ones((CH, 16), jnp.float32)
    agg_pad, cnt_pad = _sc_segment_sum(e1, dst, npad, zeros_init, zeros_cnt,
                                       ones_blk)
    agg_sum = agg_pad[:n]
    cnt = cnt_pad[:n]

    x1, x1a, x1b = _tc_node(x, agg_sum, cnt,
                            Wn1a[:d], Wn1a[d:], bn1a.reshape(1, l),
                            Wn2a, bn2a.reshape(1, l),
                            We1b[:l], We1b[l:2 * l], block=1000)

    xs1, xd1 = _sc_gather_pair2(x1a, x1b, src, dst)
    out = _tc_edge2(xs1, xd1, e1,
                    We1b[2 * l:],
                    be1b.reshape(1, l), We2b, be2b.reshape(1, l),
                    Wp1, bp1.reshape(1, l),
                    Wp2.reshape(1, l), bp2.reshape(1, 1),
                    block=2000)
    return out[:, 0]


# speedup vs baseline: 2.1836x; 1.1181x over previous
import functools

import jax
import jax.numpy as jnp
from jax import lax
from jax.experimental import pallas as pl
from jax.experimental.pallas import tpu as pltpu
from jax.experimental.pallas import tpu_sc as plsc

NC = 2
NS = 16
NW = NC * NS

CH = 80


def _sc_gather_pair2(table_s, table_d, src, dst):
    n, d = table_s.shape
    e = src.shape[0]
    per_w = e // NW
    ch = 80 if d <= 128 else 40
    nch = per_w // ch
    mesh = plsc.VectorSubcoreMesh(core_axis_name="c", subcore_axis_name="s")

    assert nch >= 4
    nb = 4
    lead = 2

    @functools.partial(
        pl.kernel, mesh=mesh,
        out_type=[jax.ShapeDtypeStruct((e, d), jnp.float32),
                  jax.ShapeDtypeStruct((e, d), jnp.float32)],
        scratch_types=([pltpu.VMEM((ch,), jnp.int32)] * nb
                       + [pltpu.VMEM((ch,), jnp.int32)] * nb
                       + [pltpu.VMEM((ch, d), jnp.float32)] * nb
                       + [pltpu.VMEM((ch, d), jnp.float32)] * nb
                       + [pltpu.SemaphoreType.DMA] * nb
                       + [pltpu.SemaphoreType.DMA] * nb),
    )
    def k(tabs_hbm, tabd_hbm, src_hbm, dst_hbm, outs_hbm, outd_hbm, *scr):
        idx_s = scr[0:nb]
        idx_d = scr[nb:2 * nb]
        rows_s = scr[2 * nb:3 * nb]
        rows_d = scr[3 * nb:4 * nb]
        gsem = scr[4 * nb:5 * nb]
        osem = scr[5 * nb:6 * nb]
        wid = lax.axis_index("s") * NC + lax.axis_index("c")
        base = wid * per_w

        def fire(ci, b):
            off = base + ci * ch
            pltpu.sync_copy(src_hbm.at[pl.ds(off, ch)], idx_s[b])
            pltpu.sync_copy(dst_hbm.at[pl.ds(off, ch)], idx_d[b])
            pltpu.async_copy(tabs_hbm.at[idx_s[b]], rows_s[b], gsem[b])
            pltpu.async_copy(tabd_hbm.at[idx_d[b]], rows_d[b], gsem[b])

        def wait_out(b):
            pltpu.make_async_copy(rows_s[b], outs_hbm.at[pl.ds(base, ch)],
                                  osem[b]).wait()
            pltpu.make_async_copy(rows_d[b], outd_hbm.at[pl.ds(base, ch)],
                                  osem[b]).wait()

        for ci in range(lead):
            fire(ci, ci)

        def step(c, b):
            fb = (b + lead) % nb
            f = c + lead

            @pl.when(c < nch)
            def _drain():
                pltpu.make_async_copy(tabs_hbm.at[idx_s[b]], rows_s[b],
                                      gsem[b]).wait()
                pltpu.make_async_copy(tabd_hbm.at[idx_d[b]], rows_d[b],
                                      gsem[b]).wait()
                off = base + c * ch
                pltpu.async_copy(rows_s[b], outs_hbm.at[pl.ds(off, ch)],
                                 osem[b])
                pltpu.async_copy(rows_d[b], outd_hbm.at[pl.ds(off, ch)],
                                 osem[b])

            @pl.when(jnp.logical_and(f < nch, f >= nb))
            def _po():
                wait_out(fb)

            @pl.when(f < nch)
            def _fire():
                fire(f, fb)

        def body(i, carry):
            for b in range(nb):
                step(i * nb + b, b)
            return carry

        lax.fori_loop(0, (nch + nb - 1) // nb, body, 0)

        for t in range(nb):
            wait_out((nch - nb + t) % nb)

    return k(table_s, table_d, src, dst)


def _sc_segment_sum(vals, dst, n, zeros_init, zeros_cnt, ones_blk):
    e, feat = vals.shape
    half = feat // NC
    per_t = e // NS
    nch = per_t // CH
    hn = n // 2
    hz = hn + 128
    zt = hz // NS
    pt = hn // NS
    assert pt % CH == 0 and zt % 8 == 0 and hn % NS == 0 and hz % NS == 0
    mesh = plsc.VectorSubcoreMesh(core_axis_name="c", subcore_axis_name="s")

    @functools.partial(
        pl.kernel, mesh=mesh,
        out_type=[jax.ShapeDtypeStruct((n, feat), jnp.float32),
                  jax.ShapeDtypeStruct((n, 16), jnp.float32)],
        scratch_types=[pltpu.VMEM((CH,), jnp.int32),
                       pltpu.VMEM((CH,), jnp.int32),
                       pltpu.VMEM((CH, half), jnp.float32),
                       pltpu.VMEM((CH, 16), jnp.float32),
                       pltpu.VMEM((8, half), jnp.float32),
                       pltpu.VMEM((8, 16), jnp.float32),
                       pltpu.VMEM((CH, 16), jnp.float32),
                       pltpu.VMEM_SHARED((hz, half), jnp.float32),
                       pltpu.VMEM_SHARED((hz, 16), jnp.float32)],
    )
    def k(vals_hbm, dst_hbm, z_hbm, zc_hbm, ones_hbm, agg_hbm, cnt_hbm,
          idx_v, idx_m, rows_v, ones_v, zbuf, cbuf, cnt_b, agg_sh, cnt_sh):
        c = lax.axis_index("c")
        s = lax.axis_index("s")
        coff = pl.multiple_of(c * half, half)

        pltpu.sync_copy(z_hbm, zbuf)
        pltpu.sync_copy(zc_hbm, cbuf)
        pltpu.sync_copy(ones_hbm, ones_v)

        for p in range(2):
            base = p * hn
            def zloop(i, carry):
                pltpu.sync_copy(zbuf, agg_sh.at[pl.ds(s * zt + i * 8, 8)])
                pltpu.sync_copy(cbuf, cnt_sh.at[pl.ds(s * zt + i * 8, 8)])
                return carry

            lax.fori_loop(0, zt // 8, zloop, 0)
            plsc.subcore_barrier()

            def body(i, carry):
                off = s * per_t + i * CH
                pltpu.sync_copy(dst_hbm.at[pl.ds(off, CH)], idx_v)
                pltpu.sync_copy(vals_hbm.at[pl.ds(off, CH),
                                            pl.ds(coff, half)], rows_v)
                for j in range(CH // 16):
                    v = idx_v[pl.ds(j * 16, 16)] - base
                    ok = jnp.logical_and(v >= 0, v < hn)
                    idx_m[pl.ds(j * 16, 16)] = jnp.where(ok, v, hn)
                pltpu.sync_copy(rows_v, agg_sh.at[idx_m], add=True)

                @pl.when(c == 0)
                def _cnt():
                    pltpu.sync_copy(ones_v, cnt_sh.at[idx_m], add=True)

                return carry

            lax.fori_loop(0, nch, body, 0)
            plsc.subcore_barrier()

            def oloop(i, carry):
                r = s * pt + i * CH
                pltpu.sync_copy(agg_sh.at[pl.ds(r, CH)], rows_v)
                pltpu.sync_copy(rows_v,
                                agg_hbm.at[pl.ds(base + r, CH),
                                           pl.ds(coff, half)])

                @pl.when(c == 0)
                def _oc():
                    pltpu.sync_copy(cnt_sh.at[pl.ds(r, CH)], cnt_b)
                    pltpu.sync_copy(cnt_b, cnt_hbm.at[pl.ds(base + r, CH)])

                return carry

            lax.fori_loop(0, pt // CH, oloop, 0)
            plsc.subcore_barrier()

    return k(vals, dst, zeros_init, zeros_cnt, ones_blk)


def _e1_body(xs, xd, ea, w1s, w1d, w1e, b1, w2, b2, out):
    h = xs[...] @ w1s[...] + xd[...] @ w1d[...] + ea[...] @ w1e[...] + b1[...]
    h = jnp.maximum(h, 0.0)
    out[...] = h @ w2[...] + b2[...] + h


def _tc_edge1(xs, xd, ea, w1s, w1d, w1e, b1, w2, b2, block):
    e, d = xs.shape
    de = ea.shape[1]
    l = w2.shape[0]
    full = lambda *shape: pl.BlockSpec(shape, lambda i: (0,) * len(shape))
    return pl.pallas_call(
        _e1_body,
        grid=(e // block,),
        in_specs=[pl.BlockSpec((block, d), lambda i: (i, 0)),
                  pl.BlockSpec((block, d), lambda i: (i, 0)),
                  pl.BlockSpec((block, de), lambda i: (i, 0)),
                  full(d, l), full(d, l), full(de, l), full(1, l),
                  full(l, l), full(1, l)],
        out_specs=pl.BlockSpec((block, l), lambda i: (i, 0)),
        out_shape=jax.ShapeDtypeStruct((e, l), jnp.float32),
    )(xs, xd, ea, w1s, w1d, w1e, b1, w2, b2)


def _node_body(x, aggs, cnt, w1x, w1a, b1, w2, b2, wbs, wbd,
               out, outa, outb):
    agg = aggs[...] / jnp.maximum(cnt[...][:, 0:1], 1.0)
    h = x[...] @ w1x[...] + agg @ w1a[...] + b1[...]
    h = jnp.maximum(h, 0.0)
    x1 = h @ w2[...] + b2[...] + h
    out[...] = x1
    outa[...] = x1 @ wbs[...]
    outb[...] = x1 @ wbd[...]


def _tc_node(x, agg_sum, cnt, w1x, w1a, b1, w2, b2, wbs, wbd, block):
    n, d = x.shape
    l = w2.shape[0]
    full = lambda *shape: pl.BlockSpec(shape, lambda i: (0,) * len(shape))
    return pl.pallas_call(
        _node_body,
        grid=(n // block,),
        in_specs=[pl.BlockSpec((block, d), lambda i: (i, 0)),
                  pl.BlockSpec((block, l), lambda i: (i, 0)),
                  pl.BlockSpec((block, 16), lambda i: (i, 0)),
                  full(d, l), full(l, l), full(1, l),
                  full(l, l), full(1, l), full(l, l), full(l, l)],
        out_specs=[pl.BlockSpec((block, l), lambda i: (i, 0))] * 3,
        out_shape=[jax.ShapeDtypeStruct((n, l), jnp.float32)] * 3,
    )(x, agg_sum, cnt, w1x, w1a, b1, w2, b2, wbs, wbd)


def _e2_body(s1, d1, e1, w1e, b1, w2, b2, wp1, bp1, wp2r, bp2, out):
    h = s1[...] + d1[...] + e1[...] @ w1e[...] + b1[...]
    h = jnp.maximum(h, 0.0)
    e2 = h @ w2[...] + b2[...] + h
    t = e2 @ wp1[...] + bp1[...]
    p = jnp.where(t >= 0.0, t, t * 0.01)
    out[...] = jnp.sum(p * wp2r[...], axis=1, keepdims=True) + bp2[...]


def _tc_edge2(s1, d1, e1, w1e, b1, w2, b2, wp1, bp1, wp2r, bp2, block):
    e, l = e1.shape
    full = lambda *shape: pl.BlockSpec(shape, lambda i: (0,) * len(shape))
    return pl.pallas_call(
        _e2_body,
        grid=(e // block,),
        in_specs=[pl.BlockSpec((block, l), lambda i: (i, 0)),
                  pl.BlockSpec((block, l), lambda i: (i, 0)),
                  pl.BlockSpec((block, l), lambda i: (i, 0)),
                  full(l, l), full(1, l),
                  full(l, l), full(1, l),
                  full(l, l), full(1, l), full(1, l), full(1, 1)],
        out_specs=pl.BlockSpec((block, 1), lambda i: (i, 0)),
        out_shape=jax.ShapeDtypeStruct((e, 1), jnp.float32),
    )(s1, d1, e1, w1e, b1, w2, b2, wp1, bp1, wp2r, bp2)


def kernel(x, edge_index, edge_attr, We1a, be1a, We2a, be2a, Wn1a, bn1a,
           Wn2a, bn2a, We1b, be1b, We2b, be2b, Wn1b, bn1b, Wn2b, bn2b,
           Wp1, bp1, Wp2, bp2):
    n, d = x.shape
    l = We2a.shape[0]
    src = edge_index[0].astype(jnp.int32)
    dst = edge_index[1].astype(jnp.int32)

    xs, xd = _sc_gather_pair2(x, x, src, dst)
    e1 = _tc_edge1(xs, xd, edge_attr,
                   We1a[:d], We1a[d:2 * d], We1a[2 * d:],
                   be1a.reshape(1, l), We2a, be2a.reshape(1, l),
                   block=2000)

    npad = ((n + 1279) // 1280) * 1280
    zeros_init = jnp.zeros((8, l // NC), jnp.float32)
    zeros_cnt = jnp.zeros((8, 16), jnp.float32)
    ones_blk = jnp.ones((CH, 16), jnp.float32)
    agg_pad, cnt_pad = _sc_segment_sum(e1, dst, npad, zeros_init, zeros_cnt,
                                       ones_blk)
    agg_sum = agg_pad[:n]
    cnt = cnt_pad[:n]

    x1, x1a, x1b = _tc_node(x, agg_sum, cnt,
                            Wn1a[:d], Wn1a[d:], bn1a.reshape(1, l),
                            Wn2a, bn2a.reshape(1, l),
                            We1b[:l], We1b[l:2 * l], block=1000)

    xs1, xd1 = _sc_gather_pair2(x1a, x1b, src, dst)
    out = _tc_edge2(xs1, xd1, e1,
                    We1b[2 * l:],
                    be1b.reshape(1, l), We2b, be2b.reshape(1, l),
                    Wp1, bp1.reshape(1, l),
                    Wp2.reshape(1, l), bp2.reshape(1, 1),
                    block=2000)
    return out[:, 0]
